# baseline (speedup 1.0000x reference)
import functools

import jax
import jax.numpy as jnp
from jax import lax
from jax.experimental import pallas as pl
from jax.experimental.pallas import tpu as pltpu
from jax.experimental.pallas import tpu_sc as plsc

N_ROWS = 16384
N_COLS = 2048
BLK = 1024
NBLK = N_ROWS // BLK
NF = float(N_ROWS)

SC_NC = 2
SC_NS = 16
SC_NW = SC_NC * SC_NS
SC_ROWS = N_ROWS // SC_NW
LANES = 16
NCHUNK = N_COLS // LANES
RBATCH = 8


def _pass1(x_ref, o_ref, var_ref, flag16_ref, acc_sum, acc_sq):
    i = pl.program_id(0)

    @pl.when(i == 0)
    def _init():
        acc_sum[...] = jnp.zeros_like(acc_sum)
        acc_sq[...] = jnp.zeros_like(acc_sq)

    x = x_ref[...]
    o_ref[...] = x
    xr = x.reshape(BLK // 8, 8, N_COLS)
    acc_sum[...] += jnp.sum(xr, axis=0)
    acc_sq[...] += jnp.sum(xr * xr, axis=0)

    @pl.when(i == NBLK - 1)
    def _finish():
        s = jnp.sum(acc_sum[...], axis=0, keepdims=True)
        q = jnp.sum(acc_sq[...], axis=0, keepdims=True)
        var = (q - s * s / NF) / (NF - 1.0)
        var_ref[...] = var
        m = (var > 0.0).astype(jnp.float32)
        cnt = jnp.sum(m, axis=1, keepdims=True)
        allk = (cnt == float(N_COLS)).astype(jnp.int32)
        flag16_ref[...] = jnp.broadcast_to(allk, (1, LANES))


def _sc_fix(x_hbm, var_hbm, flag_hbm, out_hbm, flagv, varv, idxv, rowin, rowout):
    wid = lax.axis_index("s") * SC_NC + lax.axis_index("c")
    pltpu.sync_copy(flag_hbm.at[0], flagv)
    slow = jnp.sum(flagv[...]) < 17

    @pl.when(slow)
    def _fix():
        pltpu.sync_copy(var_hbm.at[0], varv)

        def _zero(c, carry):
            idxv[pl.ds(c * LANES, LANES)] = jnp.zeros((LANES,), jnp.int32)
            return carry

        lax.fori_loop(0, NCHUNK, _zero, 0)

        def _scan(c, carry):
            v = varv[pl.ds(c * LANES, LANES)]
            m = v > 0.0
            mi = m.astype(jnp.int32)
            cs = plsc.cumsum(mi)
            pos = cs - 1 + carry
            j = jnp.arange(LANES, dtype=jnp.int32) + c * LANES
            plsc.store_scatter(idxv, [pos], j, mask=m)
            return carry + jnp.sum(mi)

        lax.fori_loop(0, NCHUNK, _scan, jnp.int32(0))

        def _rowblk(r, carry):
            base = wid * SC_ROWS + r * RBATCH
            pltpu.sync_copy(x_hbm.at[pl.ds(base, RBATCH)], rowin)

            def _chunk(c, inner):
                idx16 = idxv[pl.ds(c * LANES, LANES)]
                for rr in range(RBATCH):
                    rvec = jnp.full((LANES,), rr, jnp.int32)
                    rowout[rr, pl.ds(c * LANES, LANES)] = plsc.load_gather(
                        rowin, [rvec, idx16]
                    )
                return inner

            lax.fori_loop(0, NCHUNK, _chunk, 0)
            pltpu.sync_copy(rowout, out_hbm.at[pl.ds(base, RBATCH)])
            return carry

        lax.fori_loop(0, SC_ROWS // RBATCH, _rowblk, 0)


@functools.cache
def _sc_fix_call():
    mesh = plsc.VectorSubcoreMesh(
        core_axis_name="c", subcore_axis_name="s",
        num_cores=SC_NC, num_subcores=SC_NS,
    )
    return pl.kernel(
        _sc_fix,
        out_type=jax.ShapeDtypeStruct((N_ROWS, N_COLS), jnp.float32),
        mesh=mesh,
        scratch_types=[
            pltpu.VMEM((LANES,), jnp.int32),
            pltpu.VMEM((N_COLS,), jnp.float32),
            pltpu.VMEM((N_COLS,), jnp.int32),
            pltpu.VMEM((RBATCH, N_COLS), jnp.float32),
            pltpu.VMEM((RBATCH, N_COLS), jnp.float32),
        ],
        compiler_params=pltpu.CompilerParams(needs_layout_passes=False),
    )


def kernel(X):
    spec, var, flag16 = pl.pallas_call(
        _pass1,
        grid=(NBLK,),
        in_specs=[pl.BlockSpec((BLK, N_COLS), lambda i: (i, 0))],
        out_specs=[
            pl.BlockSpec((BLK, N_COLS), lambda i: (i, 0)),
            pl.BlockSpec((1, N_COLS), lambda i: (0, 0)),
            pl.BlockSpec((1, LANES), lambda i: (0, 0)),
        ],
        out_shape=[
            jax.ShapeDtypeStruct((N_ROWS, N_COLS), jnp.float32),
            jax.ShapeDtypeStruct((1, N_COLS), jnp.float32),
            jax.ShapeDtypeStruct((1, LANES), jnp.int32),
        ],
        scratch_shapes=[
            pltpu.VMEM((8, N_COLS), jnp.float32),
            pltpu.VMEM((8, N_COLS), jnp.float32),
        ],
        compiler_params=pltpu.CompilerParams(
            dimension_semantics=("arbitrary",),
        ),
    )(X)

    allk = flag16[0, 0] == 2
    out = lax.cond(
        allk,
        lambda: spec,
        lambda: _sc_fix_call()(X, var, flag16),
    )
    return out


# speedup vs baseline: 5.5938x; 5.5938x over previous
import functools

import jax
import jax.numpy as jnp
from jax import lax
from jax.experimental import pallas as pl
from jax.experimental.pallas import tpu as pltpu
from jax.experimental.pallas import tpu_sc as plsc

N_ROWS = 16384
N_COLS = 2048
BLK = 1024
NBLK = N_ROWS // BLK
NF = float(N_ROWS)

SC_NC = 2
SC_NS = 16
SC_NW = SC_NC * SC_NS
SC_ROWS = N_ROWS // SC_NW
LANES = 16
NCHUNK = N_COLS // LANES
RBATCH = 8


def _pass1(x_ref, o_ref, var_ref, flag16_ref, acc_sum, acc_sq):
    i = pl.program_id(0)

    @pl.when(i == 0)
    def _init():
        acc_sum[...] = jnp.zeros_like(acc_sum)
        acc_sq[...] = jnp.zeros_like(acc_sq)

    x = x_ref[...]
    o_ref[...] = x
    xr = x.reshape(BLK // 8, 8, N_COLS)
    acc_sum[...] += jnp.sum(xr, axis=0)
    acc_sq[...] += jnp.sum(xr * xr, axis=0)

    @pl.when(i == NBLK - 1)
    def _finish():
        s = jnp.sum(acc_sum[...], axis=0, keepdims=True)
        q = jnp.sum(acc_sq[...], axis=0, keepdims=True)
        var = (q - s * s / NF) / (NF - 1.0)
        var_ref[...] = var
        m = (var > 0.0).astype(jnp.float32)
        cnt = jnp.sum(m, axis=1, keepdims=True)
        allk = (cnt == float(N_COLS)).astype(jnp.int32)
        flag16_ref[...] = jnp.broadcast_to(allk, (1, LANES))


def _sc_fix(x_hbm, var_hbm, flag_hbm, out_hbm, flagv, varv, idxv, rowin, rowout):
    wid = lax.axis_index("s") * SC_NC + lax.axis_index("c")
    pltpu.sync_copy(flag_hbm.at[0], flagv)
    slow = jnp.sum(flagv[...]) == 0

    @pl.when(slow)
    def _fix():
        pltpu.sync_copy(var_hbm.at[0], varv)

        def _zero(c, carry):
            idxv[pl.ds(c * LANES, LANES)] = jnp.zeros((LANES,), jnp.int32)
            return carry

        lax.fori_loop(0, NCHUNK, _zero, 0)

        def _scan(c, carry):
            v = varv[pl.ds(c * LANES, LANES)]
            m = v > 0.0
            mi = m.astype(jnp.int32)
            cs = plsc.cumsum(mi)
            pos = cs - 1 + carry
            j = jnp.arange(LANES, dtype=jnp.int32) + c * LANES
            plsc.store_scatter(idxv, [pos], j, mask=m)
            return carry + jnp.sum(mi)

        lax.fori_loop(0, NCHUNK, _scan, jnp.int32(0))

        def _rowblk(r, carry):
            base = wid * SC_ROWS + r * RBATCH
            pltpu.sync_copy(x_hbm.at[pl.ds(base, RBATCH)], rowin)

            def _chunk(c, inner):
                idx16 = idxv[pl.ds(c * LANES, LANES)]
                for rr in range(RBATCH):
                    rvec = jnp.full((LANES,), rr, jnp.int32)
                    rowout[rr, pl.ds(c * LANES, LANES)] = plsc.load_gather(
                        rowin, [rvec, idx16]
                    )
                return inner

            lax.fori_loop(0, NCHUNK, _chunk, 0)
            pltpu.sync_copy(rowout, out_hbm.at[pl.ds(base, RBATCH)])
            return carry

        lax.fori_loop(0, SC_ROWS // RBATCH, _rowblk, 0)


@functools.cache
def _sc_fix_call():
    mesh = plsc.VectorSubcoreMesh(
        core_axis_name="c", subcore_axis_name="s",
        num_cores=SC_NC, num_subcores=SC_NS,
    )
    return pl.kernel(
        _sc_fix,
        out_type=jax.ShapeDtypeStruct((N_ROWS, N_COLS), jnp.float32),
        mesh=mesh,
        scratch_types=[
            pltpu.VMEM((LANES,), jnp.int32),
            pltpu.VMEM((N_COLS,), jnp.float32),
            pltpu.VMEM((N_COLS,), jnp.int32),
            pltpu.VMEM((RBATCH, N_COLS), jnp.float32),
            pltpu.VMEM((RBATCH, N_COLS), jnp.float32),
        ],
        compiler_params=pltpu.CompilerParams(needs_layout_passes=False),
    )


def kernel(X):
    spec, var, flag16 = pl.pallas_call(
        _pass1,
        grid=(NBLK,),
        in_specs=[pl.BlockSpec((BLK, N_COLS), lambda i: (i, 0))],
        out_specs=[
            pl.BlockSpec((BLK, N_COLS), lambda i: (i, 0)),
            pl.BlockSpec((1, N_COLS), lambda i: (0, 0)),
            pl.BlockSpec((1, LANES), lambda i: (0, 0)),
        ],
        out_shape=[
            jax.ShapeDtypeStruct((N_ROWS, N_COLS), jnp.float32),
            jax.ShapeDtypeStruct((1, N_COLS), jnp.float32),
            jax.ShapeDtypeStruct((1, LANES), jnp.int32),
        ],
        scratch_shapes=[
            pltpu.VMEM((8, N_COLS), jnp.float32),
            pltpu.VMEM((8, N_COLS), jnp.float32),
        ],
        compiler_params=pltpu.CompilerParams(
            dimension_semantics=("arbitrary",),
        ),
    )(X)

    allk = flag16[0, 0] == 1
    out = lax.cond(
        allk,
        lambda: spec,
        lambda: _sc_fix_call()(X, var, flag16),
    )
    return out
